# Initial kernel scaffold; baseline (speedup 1.0000x reference)
import jax
import jax.numpy as jnp
from jax.experimental import pallas as pl


def kernel(x, edge_index, batch, eval_edges, lin_W0, lin_b0, agg_W0, agg_b0, lin_W1, lin_b1, agg_W1, agg_b1, pW1, pb1, pW2, pb2):
    raise NotImplementedError("write your pallas kernel here")


# speedup vs baseline: 6.4079x; 6.4079x over previous
import functools

import jax
import jax.numpy as jnp
from jax import lax
from jax.experimental import pallas as pl
from jax.experimental.pallas import tpu as pltpu
from jax.experimental.pallas import tpu_sc as plsc

N = 10000
D = 128
H = 128
E = 320000
EV = 10000
OUT = 2

NC = 2
NS = 16
NW = NC * NS
EPT = E // NW
K = 80
NCH = EPT // K
NP = 10240
RPT = NP // NS

EVP = 10240
KE = 80
ECH = EVP // (NW * KE)
TW = 16


def _tc_lin_relu(h_ref, w_ref, b_ref, o_ref):
    o_ref[...] = jnp.maximum(
        jnp.dot(h_ref[...], w_ref[...], preferred_element_type=jnp.float32)
        + b_ref[...], 0.0)


def _tc_agg(p_ref, h_ref, wa_ref, wh_ref, b_ref, o_ref):
    aggr = p_ref[0, :N, :] + p_ref[1, :N, :]
    out = (jnp.dot(aggr, wa_ref[...], preferred_element_type=jnp.float32)
           + jnp.dot(h_ref[...], wh_ref[...], preferred_element_type=jnp.float32)
           + b_ref[...])
    out = jnp.maximum(out, 0.0)
    nrm = jnp.maximum(jnp.sqrt(jnp.sum(out * out, axis=1, keepdims=True)), 1e-12)
    o_ref[...] = out / nrm


def _tc_head(g_ref, pw1_ref, pw2_ref, pb1_ref, pb2_ref, o_ref):
    small = jnp.dot(pw1_ref[...], pw2_ref[...],
                    preferred_element_type=jnp.float32)
    c = jnp.dot(pb1_ref[...], pw2_ref[...],
                preferred_element_type=jnp.float32) + pb2_ref[...]
    z = (jnp.dot(g_ref[0], small[:H], preferred_element_type=jnp.float32)
         + jnp.dot(g_ref[1], small[H:], preferred_element_type=jnp.float32)
         + c)
    z0 = z[:, 0:1]
    z1 = z[:, 1:2]
    m = jnp.maximum(z0, z1)
    lse = m + jnp.log(jnp.exp(z0 - m) + jnp.exp(z1 - m))
    o_ref[...] = jnp.concatenate([z0 - lse, z1 - lse], axis=1)


def _sc_spmm(t_hbm, src_hbm, dst_hbm, zero_hbm, out_hbm,
             src_v, dst_v, rows_v, acc_sh, sem):
    ci = lax.axis_index("c")
    si = lax.axis_index("s")
    w = ci * NS + si
    pltpu.sync_copy(src_hbm.at[w], src_v)
    pltpu.sync_copy(dst_hbm.at[w], dst_v)
    pltpu.sync_copy(zero_hbm, acc_sh.at[pl.ds(si * RPT, RPT)])
    plsc.subcore_barrier()

    @pl.loop(0, NCH)
    def _(c):
        pltpu.async_copy(t_hbm.at[src_v.at[c]], rows_v, sem).wait()
        pltpu.sync_copy(rows_v, acc_sh.at[dst_v.at[c]], add=True)

    plsc.subcore_barrier()
    pltpu.sync_copy(acc_sh.at[pl.ds(si * RPT, RPT)],
                    out_hbm.at[ci].at[pl.ds(si * RPT, RPT)])


def _sc_eval_gather(tab_hbm, e0_hbm, e1_hbm, out_hbm, i0_v, i1_v, rows_v, sem):
    ci = lax.axis_index("c")
    si = lax.axis_index("s")
    w = ci * NS + si
    pltpu.sync_copy(e0_hbm.at[w], i0_v)
    pltpu.sync_copy(e1_hbm.at[w], i1_v)

    @pl.loop(0, ECH)
    def _(c):
        base = w * (ECH * KE) + c * KE
        pltpu.async_copy(tab_hbm.at[i0_v.at[c]], rows_v, sem).wait()
        pltpu.sync_copy(rows_v, out_hbm.at[0].at[pl.ds(base, KE)])
        pltpu.async_copy(tab_hbm.at[i1_v.at[c]], rows_v, sem).wait()
        pltpu.sync_copy(rows_v, out_hbm.at[1].at[pl.ds(base, KE)])


def _vmesh():
    return plsc.VectorSubcoreMesh(core_axis_name="c", subcore_axis_name="s")


def kernel(x, edge_index, batch, eval_edges, lin_W0, lin_b0, agg_W0, agg_b0,
           lin_W1, lin_b1, agg_W1, agg_b1, pW1, pb1, pW2, pb2):
    del batch

    f32 = jnp.float32
    src = edge_index[0].reshape(NW, NCH, K)
    dst = edge_index[1].reshape(NW, NCH, K)
    zero_rows = jnp.zeros((RPT, H), f32)

    pad = jnp.zeros((EVP - EV,), jnp.int32)
    e0 = jnp.concatenate([eval_edges[0], pad]).reshape(NW, ECH, KE)
    e1 = jnp.concatenate([eval_edges[1], pad]).reshape(NW, ECH, KE)

    lin_relu = pl.pallas_call(
        _tc_lin_relu, out_shape=jax.ShapeDtypeStruct((N, H), f32))
    agg = pl.pallas_call(
        _tc_agg, out_shape=jax.ShapeDtypeStruct((N, H), f32))
    head = pl.pallas_call(
        _tc_head, out_shape=jax.ShapeDtypeStruct((EVP, OUT), f32))

    spmm = functools.partial(
        pl.kernel,
        out_type=jax.ShapeDtypeStruct((NC, NP, H), f32),
        mesh=_vmesh(),
        scratch_types=[
            pltpu.VMEM((NCH, K), jnp.int32),
            pltpu.VMEM((NCH, K), jnp.int32),
            pltpu.VMEM((K, H), f32),
            pltpu.VMEM_SHARED((NP, H), f32),
            pltpu.SemaphoreType.DMA,
        ],
    )(_sc_spmm)

    eval_gather = functools.partial(
        pl.kernel,
        out_type=jax.ShapeDtypeStruct((2, EVP, H), f32),
        mesh=_vmesh(),
        scratch_types=[
            pltpu.VMEM((ECH, KE), jnp.int32),
            pltpu.VMEM((ECH, KE), jnp.int32),
            pltpu.VMEM((KE, H), f32),
            pltpu.SemaphoreType.DMA,
        ],
    )(_sc_eval_gather)

    h = x
    for lW, lb, aW, ab in ((lin_W0, lin_b0, agg_W0, agg_b0),
                           (lin_W1, lin_b1, agg_W1, agg_b1)):
        t = lin_relu(h, lW, lb.reshape(1, H))
        parts = spmm(t, src, dst, zero_rows)
        h = agg(parts, h, aW[:H], aW[H:], ab.reshape(1, H))

    g = eval_gather(h, e0, e1)
    out = head(g, pW1, pW2, pb1.reshape(1, H), pb2.reshape(1, OUT))
    return out[:EV]


# speedup vs baseline: 9.4645x; 1.4770x over previous
import functools

import jax
import jax.numpy as jnp
from jax import lax
from jax.experimental import pallas as pl
from jax.experimental.pallas import tpu as pltpu
from jax.experimental.pallas import tpu_sc as plsc

N = 10000
D = 128
H = 128
E = 320000
EV = 10000
OUT = 2

NC = 2
NS = 16
NW = NC * NS
EPT = E // NW
K = 80
NCH = EPT // K
NP = 10240
RPT = NP // NS

EVP = 10240
KE = 80
ECH = EVP // (NW * KE)
TW = 16


def _tc_lin_relu(h_ref, w_ref, b_ref, o_ref):
    o_ref[...] = jnp.maximum(
        jnp.dot(h_ref[...], w_ref[...], preferred_element_type=jnp.float32)
        + b_ref[...], 0.0)


def _tc_agg(p_ref, h_ref, wa_ref, wh_ref, b_ref, o_ref):
    aggr = p_ref[0, :N, :] + p_ref[1, :N, :]
    out = (jnp.dot(aggr, wa_ref[...], preferred_element_type=jnp.float32)
           + jnp.dot(h_ref[...], wh_ref[...], preferred_element_type=jnp.float32)
           + b_ref[...])
    out = jnp.maximum(out, 0.0)
    nrm = jnp.maximum(jnp.sqrt(jnp.sum(out * out, axis=1, keepdims=True)), 1e-12)
    o_ref[...] = out / nrm


def _tc_head(g_ref, pw1_ref, pw2_ref, pb1_ref, pb2_ref, o_ref):
    small = jnp.dot(pw1_ref[...], pw2_ref[...],
                    preferred_element_type=jnp.float32)
    c = jnp.dot(pb1_ref[...], pw2_ref[...],
                preferred_element_type=jnp.float32) + pb2_ref[...]
    z = (jnp.dot(g_ref[0], small[:H], preferred_element_type=jnp.float32)
         + jnp.dot(g_ref[1], small[H:], preferred_element_type=jnp.float32)
         + c)
    z0 = z[:, 0:1]
    z1 = z[:, 1:2]
    m = jnp.maximum(z0, z1)
    lse = m + jnp.log(jnp.exp(z0 - m) + jnp.exp(z1 - m))
    o_ref[...] = jnp.concatenate([z0 - lse, z1 - lse], axis=1)


def _sc_spmm(t_hbm, src_hbm, dst_hbm, zero_hbm, out_hbm,
             src_v, dst_v, buf0, buf1, acc_sh, sem0, sem1):
    ci = lax.axis_index("c")
    si = lax.axis_index("s")
    w = ci * NS + si
    pltpu.sync_copy(src_hbm.at[w], src_v)
    pltpu.sync_copy(dst_hbm.at[w], dst_v)
    pltpu.sync_copy(zero_hbm, acc_sh.at[pl.ds(si * RPT, RPT)])
    plsc.subcore_barrier()

    def wait0():
        pltpu.make_async_copy(t_hbm.at[pl.ds(0, K)], buf0, sem0).wait()

    pltpu.async_copy(t_hbm.at[src_v.at[pl.ds(0, K)]], buf0, sem0)

    @pl.loop(0, NCH - 1, step=2)
    def _(c):
        d1 = pltpu.async_copy(t_hbm.at[src_v.at[pl.ds((c + 1) * K, K)]], buf1, sem1)
        wait0()
        pltpu.sync_copy(buf0, acc_sh.at[dst_v.at[c]], add=True)
        pltpu.async_copy(t_hbm.at[src_v.at[pl.ds((c + 2) * K, K)]], buf0, sem0)
        d1.wait()
        pltpu.sync_copy(buf1, acc_sh.at[dst_v.at[c + 1]], add=True)

    wait0()
    pltpu.sync_copy(buf0, acc_sh.at[dst_v.at[NCH - 1]], add=True)
    plsc.subcore_barrier()
    pltpu.sync_copy(acc_sh.at[pl.ds(si * RPT, RPT)],
                    out_hbm.at[ci].at[pl.ds(si * RPT, RPT)])


def _sc_eval_gather(tab_hbm, e0_hbm, e1_hbm, out_hbm, i0_v, i1_v,
                    rows0, rows1, sem0, sem1):
    ci = lax.axis_index("c")
    si = lax.axis_index("s")
    w = ci * NS + si
    pltpu.async_copy(e0_hbm.at[w], i0_v, sem0)
    pltpu.async_copy(e1_hbm.at[w], i1_v, sem1)
    pltpu.make_async_copy(e0_hbm.at[w], i0_v, sem0).wait()
    pltpu.make_async_copy(e1_hbm.at[w], i1_v, sem1).wait()

    @pl.loop(0, ECH)
    def _(c):
        base = w * (ECH * KE) + c * KE
        d0 = pltpu.async_copy(tab_hbm.at[i0_v.at[c]], rows0, sem0)
        d1 = pltpu.async_copy(tab_hbm.at[i1_v.at[c]], rows1, sem1)
        d0.wait()
        pltpu.sync_copy(rows0, out_hbm.at[0].at[pl.ds(base, KE)])
        d1.wait()
        pltpu.sync_copy(rows1, out_hbm.at[1].at[pl.ds(base, KE)])


def _vmesh():
    return plsc.VectorSubcoreMesh(core_axis_name="c", subcore_axis_name="s")


def kernel(x, edge_index, batch, eval_edges, lin_W0, lin_b0, agg_W0, agg_b0,
           lin_W1, lin_b1, agg_W1, agg_b1, pW1, pb1, pW2, pb2):
    del batch

    f32 = jnp.float32
    src = edge_index[0].reshape(NW, EPT)
    dst = edge_index[1].reshape(NW, NCH, K)
    zero_rows = jnp.zeros((RPT, H), f32)

    pad = jnp.zeros((EVP - EV,), jnp.int32)
    e0 = jnp.concatenate([eval_edges[0], pad]).reshape(NW, ECH, KE)
    e1 = jnp.concatenate([eval_edges[1], pad]).reshape(NW, ECH, KE)

    lin_relu = pl.pallas_call(
        _tc_lin_relu, out_shape=jax.ShapeDtypeStruct((N, H), f32))
    agg = pl.pallas_call(
        _tc_agg, out_shape=jax.ShapeDtypeStruct((N, H), f32))
    head = pl.pallas_call(
        _tc_head, out_shape=jax.ShapeDtypeStruct((EVP, OUT), f32))

    spmm = functools.partial(
        pl.kernel,
        out_type=jax.ShapeDtypeStruct((NC, NP, H), f32),
        mesh=_vmesh(),
        scratch_types=[
            pltpu.VMEM((EPT,), jnp.int32),
            pltpu.VMEM((NCH, K), jnp.int32),
            pltpu.VMEM((K, H), f32),
            pltpu.VMEM((K, H), f32),
            pltpu.VMEM_SHARED((NP, H), f32),
            pltpu.SemaphoreType.DMA,
            pltpu.SemaphoreType.DMA,
        ],
    )(_sc_spmm)

    eval_gather = functools.partial(
        pl.kernel,
        out_type=jax.ShapeDtypeStruct((2, EVP, H), f32),
        mesh=_vmesh(),
        scratch_types=[
            pltpu.VMEM((ECH, KE), jnp.int32),
            pltpu.VMEM((ECH, KE), jnp.int32),
            pltpu.VMEM((KE, H), f32),
            pltpu.VMEM((KE, H), f32),
            pltpu.SemaphoreType.DMA,
            pltpu.SemaphoreType.DMA,
        ],
    )(_sc_eval_gather)

    h = x
    for lW, lb, aW, ab in ((lin_W0, lin_b0, agg_W0, agg_b0),
                           (lin_W1, lin_b1, agg_W1, agg_b1)):
        t = lin_relu(h, lW, lb.reshape(1, H))
        parts = spmm(t, src, dst, zero_rows)
        h = agg(parts, h, aW[:H], aW[H:], ab.reshape(1, H))

    g = eval_gather(h, e0, e1)
    out = head(g, pW1, pW2, pb1.reshape(1, H), pb2.reshape(1, OUT))
    return out[:EV]
